# baseline (speedup 1.0000x reference)
import functools

import jax
import jax.numpy as jnp
from jax.experimental import pallas as pl

N_BLK = 2048


def _mmoe_kernel(x_ref, wc_ref, bc_ref, ac_ref, wg_ref, bg_ref, gsum_ref,
                 gexp_ref, out_ref, *, n_experts, n_tasks, units):
    xb = x_ref[...].astype(jnp.bfloat16)
    pre = jnp.dot(xb, wc_ref[...], preferred_element_type=jnp.float32)
    pre = pre + bc_ref[...]
    eo = jnp.where(pre > 0, pre, ac_ref[...] * pre)
    gl = jnp.dot(xb, wg_ref[...], preferred_element_type=jnp.float32)
    gl = gl + bg_ref[...]
    m = jnp.max(gl, axis=1, keepdims=True)
    e8 = jnp.exp(gl - m)
    denom = jnp.dot(e8.astype(jnp.bfloat16), gsum_ref[...],
                    preferred_element_type=jnp.float32)
    g8 = e8 / denom
    gwide = jnp.dot(g8.astype(jnp.bfloat16), gexp_ref[...],
                    preferred_element_type=jnp.float32)
    outs = []
    for t in range(n_tasks):
        acc = gwide[:, t * n_experts * units:t * n_experts * units + units] \
            * eo[:, 0:units]
        for e in range(1, n_experts):
            acc = acc + (gwide[:, (t * n_experts + e) * units:
                                (t * n_experts + e + 1) * units]
                         * eo[:, e * units:(e + 1) * units])
        outs.append(acc)
    out_ref[...] = jnp.concatenate(outs, axis=1)


def kernel(inputs, W_expert, b_expert, alpha, W_gate, b_gate):
    n, d = inputs.shape
    n_experts, _, units = W_expert.shape
    n_tasks = W_gate.shape[0]
    te = n_tasks * n_experts
    wc = W_expert.transpose(1, 0, 2).reshape(d, n_experts * units)
    wc = wc.astype(jnp.bfloat16)
    bc = b_expert.reshape(1, n_experts * units)
    ac = alpha.reshape(1, n_experts * units)
    wg = W_gate.transpose(1, 0, 2).reshape(d, te).astype(jnp.bfloat16)
    bg = b_gate.reshape(1, te)
    task_of = jnp.arange(te) // n_experts
    gsum = (task_of[:, None] == task_of[None, :]).astype(jnp.bfloat16)
    gexp = jnp.repeat(jnp.eye(te, dtype=jnp.bfloat16), units, axis=1)

    grid = (n // N_BLK,)
    out3d = pl.pallas_call(
        functools.partial(_mmoe_kernel, n_experts=n_experts,
                          n_tasks=n_tasks, units=units),
        grid=grid,
        in_specs=[
            pl.BlockSpec((N_BLK, d), lambda i: (i, 0)),
            pl.BlockSpec(wc.shape, lambda i: (0, 0)),
            pl.BlockSpec(bc.shape, lambda i: (0, 0)),
            pl.BlockSpec(ac.shape, lambda i: (0, 0)),
            pl.BlockSpec(wg.shape, lambda i: (0, 0)),
            pl.BlockSpec(bg.shape, lambda i: (0, 0)),
            pl.BlockSpec(gsum.shape, lambda i: (0, 0)),
            pl.BlockSpec(gexp.shape, lambda i: (0, 0)),
        ],
        out_specs=pl.BlockSpec((N_BLK, n_tasks * units), lambda i: (i, 0)),
        out_shape=jax.ShapeDtypeStruct((n, n_tasks * units), jnp.float32),
    )(inputs, wc, bc, ac, wg, bg, gsum, gexp)
    return out3d.reshape(n, n_tasks, units)


# speedup vs baseline: 1.3262x; 1.3262x over previous
import functools

import jax
import jax.numpy as jnp
import numpy as np
from jax.experimental import pallas as pl

N_BLK = 2048


def _mmoe_kernel(x_ref, wc_ref, bc_ref, ac_ref, wg_ref, bg_ref, gsum_ref,
                 gexp_ref, out_ref, *, n_experts, n_tasks, units):
    xb = x_ref[...].astype(jnp.bfloat16)
    pre = jnp.dot(xb, wc_ref[...], preferred_element_type=jnp.float32)
    pre = pre + bc_ref[...]
    eo = jnp.where(pre > 0, pre, ac_ref[...] * pre)
    gl = jnp.dot(xb, wg_ref[...], preferred_element_type=jnp.float32)
    gl = gl + bg_ref[...]
    m = jnp.max(gl, axis=1, keepdims=True)
    e8 = jnp.exp(gl - m)
    denom = jnp.dot(e8.astype(jnp.bfloat16), gsum_ref[...],
                    preferred_element_type=jnp.float32)
    g8 = e8 / denom
    gwide = jnp.dot(g8.astype(jnp.bfloat16), gexp_ref[...],
                    preferred_element_type=jnp.float32)
    for t in range(n_tasks):
        acc = gwide[:, t * n_experts * units:t * n_experts * units + units] \
            * eo[:, 0:units]
        for e in range(1, n_experts):
            acc = acc + (gwide[:, (t * n_experts + e) * units:
                                (t * n_experts + e + 1) * units]
                         * eo[:, e * units:(e + 1) * units])
        out_ref[:, t, :] = acc


def kernel(inputs, W_expert, b_expert, alpha, W_gate, b_gate):
    n, d = inputs.shape
    n_experts, _, units = W_expert.shape
    n_tasks = W_gate.shape[0]
    te = n_tasks * n_experts
    wc = W_expert.transpose(1, 0, 2).reshape(d, n_experts * units)
    wc = wc.astype(jnp.bfloat16)
    bc = b_expert.reshape(1, n_experts * units)
    ac = alpha.reshape(1, n_experts * units)
    wg = W_gate.transpose(1, 0, 2).reshape(d, te).astype(jnp.bfloat16)
    bg = b_gate.reshape(1, te)
    task_of = np.arange(te) // n_experts
    gsum = jnp.asarray((task_of[:, None] == task_of[None, :]),
                       dtype=jnp.bfloat16)
    gexp = jnp.asarray(np.repeat(np.eye(te), units, axis=1),
                       dtype=jnp.bfloat16)

    grid = (n // N_BLK,)
    out3d = pl.pallas_call(
        functools.partial(_mmoe_kernel, n_experts=n_experts,
                          n_tasks=n_tasks, units=units),
        grid=grid,
        in_specs=[
            pl.BlockSpec((N_BLK, d), lambda i: (i, 0)),
            pl.BlockSpec(wc.shape, lambda i: (0, 0)),
            pl.BlockSpec(bc.shape, lambda i: (0, 0)),
            pl.BlockSpec(ac.shape, lambda i: (0, 0)),
            pl.BlockSpec(wg.shape, lambda i: (0, 0)),
            pl.BlockSpec(bg.shape, lambda i: (0, 0)),
            pl.BlockSpec(gsum.shape, lambda i: (0, 0)),
            pl.BlockSpec(gexp.shape, lambda i: (0, 0)),
        ],
        out_specs=pl.BlockSpec((N_BLK, n_tasks, units), lambda i: (i, 0, 0)),
        out_shape=jax.ShapeDtypeStruct((n, n_tasks, units), jnp.float32),
    )(inputs, wc, bc, ac, wg, bg, gsum, gexp)
    return out3d


# speedup vs baseline: 1.3300x; 1.0029x over previous
import functools

import jax
import jax.numpy as jnp
import numpy as np
from jax.experimental import pallas as pl
from jax.experimental.pallas import tpu as pltpu

N_BLK = 4096


def _mmoe_kernel(x_ref, wc_ref, bc_ref, ac_ref, wg_ref, bg_ref, gsum_ref,
                 gexp_ref, out_ref, *, n_experts, n_tasks, units):
    xb = x_ref[...].astype(jnp.bfloat16)
    pre = jnp.dot(xb, wc_ref[...], preferred_element_type=jnp.float32)
    pre = pre + bc_ref[...]
    eo = jnp.where(pre > 0, pre, ac_ref[...] * pre)
    gl = jnp.dot(xb, wg_ref[...], preferred_element_type=jnp.float32)
    gl = gl + bg_ref[...]
    m = jnp.max(gl, axis=1, keepdims=True)
    e8 = jnp.exp(gl - m)
    denom = jnp.dot(e8.astype(jnp.bfloat16), gsum_ref[...],
                    preferred_element_type=jnp.float32)
    g8 = e8 / denom
    gwide = jnp.dot(g8.astype(jnp.bfloat16), gexp_ref[...],
                    preferred_element_type=jnp.float32)
    for t in range(n_tasks):
        acc = gwide[:, t * n_experts * units:t * n_experts * units + units] \
            * eo[:, 0:units]
        for e in range(1, n_experts):
            acc = acc + (gwide[:, (t * n_experts + e) * units:
                                (t * n_experts + e + 1) * units]
                         * eo[:, e * units:(e + 1) * units])
        out_ref[:, t, :] = acc


def kernel(inputs, W_expert, b_expert, alpha, W_gate, b_gate):
    n, d = inputs.shape
    n_experts, _, units = W_expert.shape
    n_tasks = W_gate.shape[0]
    te = n_tasks * n_experts
    wc = jnp.concatenate([W_expert[e] for e in range(n_experts)],
                         axis=1).astype(jnp.bfloat16)
    bc = b_expert.reshape(1, n_experts * units)
    ac = alpha.reshape(1, n_experts * units)
    wg = W_gate.transpose(1, 0, 2).reshape(d, te).astype(jnp.bfloat16)
    bg = b_gate.reshape(1, te)
    task_of = np.arange(te) // n_experts
    gsum = jnp.asarray((task_of[:, None] == task_of[None, :]),
                       dtype=jnp.bfloat16)
    gexp = jnp.asarray(np.repeat(np.eye(te), units, axis=1),
                       dtype=jnp.bfloat16)

    grid = (n // N_BLK,)
    out3d = pl.pallas_call(
        functools.partial(_mmoe_kernel, n_experts=n_experts,
                          n_tasks=n_tasks, units=units),
        grid=grid,
        in_specs=[
            pl.BlockSpec((N_BLK, d), lambda i: (i, 0)),
            pl.BlockSpec(wc.shape, lambda i: (0, 0)),
            pl.BlockSpec(bc.shape, lambda i: (0, 0)),
            pl.BlockSpec(ac.shape, lambda i: (0, 0)),
            pl.BlockSpec(wg.shape, lambda i: (0, 0)),
            pl.BlockSpec(bg.shape, lambda i: (0, 0)),
            pl.BlockSpec(gsum.shape, lambda i: (0, 0)),
            pl.BlockSpec(gexp.shape, lambda i: (0, 0)),
        ],
        out_specs=pl.BlockSpec((N_BLK, n_tasks, units), lambda i: (i, 0, 0)),
        out_shape=jax.ShapeDtypeStruct((n, n_tasks, units), jnp.float32),
        compiler_params=pltpu.CompilerParams(
            dimension_semantics=("parallel",)),
    )(inputs, wc, bc, ac, wg, bg, gsum, gexp)
    return out3d


# speedup vs baseline: 1.3331x; 1.0024x over previous
import functools

import jax
import jax.numpy as jnp
import numpy as np
from jax.experimental import pallas as pl
from jax.experimental.pallas import tpu as pltpu

N_BLK = 4096


def _mmoe_kernel(x_ref, wc_ref, bc_ref, ac_ref, wg_ref, bg_ref, gsum_ref,
                 gexp_ref, out_ref, *, n_experts, n_tasks, units):
    xb = x_ref[...].astype(jnp.bfloat16)
    pre = jnp.dot(xb, wc_ref[...], preferred_element_type=jnp.float32)
    pre = pre + bc_ref[...]
    eo = jnp.where(pre > 0, pre, ac_ref[...] * pre)
    gl = jnp.dot(xb, wg_ref[...], preferred_element_type=jnp.float32)
    gl = gl + bg_ref[...]
    m = jnp.max(gl, axis=1, keepdims=True)
    e8 = jnp.exp(gl - m)
    denom = jnp.dot(e8.astype(jnp.bfloat16), gsum_ref[...],
                    preferred_element_type=jnp.float32)
    g8 = e8 / denom
    gwide = jnp.dot(g8.astype(jnp.bfloat16), gexp_ref[...],
                    preferred_element_type=jnp.float32)
    for t in range(n_tasks):
        acc = gwide[:, t * n_experts * units:t * n_experts * units + units] \
            * eo[:, 0:units]
        for e in range(1, n_experts):
            acc = acc + (gwide[:, (t * n_experts + e) * units:
                                (t * n_experts + e + 1) * units]
                         * eo[:, e * units:(e + 1) * units])
        out_ref[:, t, :] = acc


def kernel(inputs, W_expert, b_expert, alpha, W_gate, b_gate):
    n, d = inputs.shape
    n_experts, _, units = W_expert.shape
    n_tasks = W_gate.shape[0]
    te = n_tasks * n_experts
    wc = W_expert.transpose(1, 0, 2).reshape(d, n_experts * units)
    wc = wc.astype(jnp.bfloat16)
    bc = b_expert.reshape(1, n_experts * units)
    ac = alpha.reshape(1, n_experts * units)
    wg = W_gate.transpose(1, 0, 2).reshape(d, te).astype(jnp.bfloat16)
    bg = b_gate.reshape(1, te)
    task_of = np.arange(te) // n_experts
    gsum = jnp.asarray((task_of[:, None] == task_of[None, :]),
                       dtype=jnp.bfloat16)
    gexp = jnp.asarray(np.repeat(np.eye(te), units, axis=1),
                       dtype=jnp.bfloat16)

    grid = (n // N_BLK,)
    out3d = pl.pallas_call(
        functools.partial(_mmoe_kernel, n_experts=n_experts,
                          n_tasks=n_tasks, units=units),
        grid=grid,
        in_specs=[
            pl.BlockSpec((N_BLK, d), lambda i: (i, 0)),
            pl.BlockSpec(wc.shape, lambda i: (0, 0)),
            pl.BlockSpec(bc.shape, lambda i: (0, 0)),
            pl.BlockSpec(ac.shape, lambda i: (0, 0)),
            pl.BlockSpec(wg.shape, lambda i: (0, 0)),
            pl.BlockSpec(bg.shape, lambda i: (0, 0)),
            pl.BlockSpec(gsum.shape, lambda i: (0, 0)),
            pl.BlockSpec(gexp.shape, lambda i: (0, 0)),
        ],
        out_specs=pl.BlockSpec((N_BLK, n_tasks, units), lambda i: (i, 0, 0)),
        out_shape=jax.ShapeDtypeStruct((n, n_tasks, units), jnp.float32),
        compiler_params=pltpu.CompilerParams(
            dimension_semantics=("parallel",)),
    )(inputs, wc, bc, ac, wg, bg, gsum, gexp)
    return out3d
